# baseline (speedup 1.0000x reference)
import dataclasses
import functools

import jax
import jax.numpy as jnp
from jax import lax
from jax.experimental import pallas as pl
from jax.experimental.pallas import tpu as pltpu
from jax.experimental.pallas import tpu_sc as plsc

N = 10000
E = 320000
DIN = 128
HID = 64
H1 = 8
OUT = 10
G = 64

NC = 2
NS = 16
NW = NC * NS
BE = 125
EPW = E // NW
NB = EPW // BE
IR = EPW // BE
WD = 80
CH = 640
ST = 624
ZR = 32


_PERM64 = [c for chunk in (0, 32) for i in range(16)
           for c in (chunk + i, chunk + 16 + i)]


def _permute_head_cols(w, heads):
    k = w.shape[0]
    wr = w.reshape(k, heads, 64)
    return wr[:, :, jnp.array(_PERM64, jnp.int32)].reshape(k, heads * 64)


def _make_matmul_grouped_kernel(ngroups, bn):
    def _kernel(x_ref, w_ref, o_ref):
        res = jnp.dot(x_ref[...], w_ref[...],
                      preferred_element_type=jnp.float32)
        for g in range(ngroups):
            o_ref[g] = res[:, g * bn:(g + 1) * bn].astype(jnp.bfloat16)
    return _kernel


def _pallas_matmul_grouped(x, w, ngroups):
    m, k = x.shape
    _, n = w.shape
    bn = n // ngroups
    bm = 2000
    return pl.pallas_call(
        _make_matmul_grouped_kernel(ngroups, bn),
        grid=(m // bm,),
        in_specs=[pl.BlockSpec((bm, k), lambda i: (i, 0)),
                  pl.BlockSpec((k, n), lambda i: (0, 0))],
        out_specs=pl.BlockSpec((ngroups, bm, bn), lambda i: (0, i, 0)),
        out_shape=jax.ShapeDtypeStruct((ngroups, m, bn), jnp.bfloat16),
    )(x, w)


def _edge_pass(ngroups):
    mesh = plsc.VectorSubcoreMesh(core_axis_name="c", subcore_axis_name="s")
    cp = pltpu.CompilerParams()
    if "needs_layout_passes" in pltpu.CompilerParams.__dataclass_fields__:
        cp = dataclasses.replace(cp, needs_layout_passes=False)
    if "use_tc_tiling_on_sc" in pltpu.CompilerParams.__dataclass_fields__:
        cp = dataclasses.replace(cp, use_tc_tiling_on_sc=False)

    @functools.partial(
        pl.kernel,
        compiler_params=cp,
        out_type=jax.ShapeDtypeStruct((ngroups, NC, N, WD), jnp.float32),
        mesh=mesh,
        scratch_types=[
            pltpu.VMEM((IR, BE), jnp.int32),
            pltpu.VMEM((IR, BE), jnp.int32),
            pltpu.VMEM((BE, HID), jnp.bfloat16),
            pltpu.VMEM((BE, HID), jnp.bfloat16),
            pltpu.VMEM((BE, HID), jnp.bfloat16),
            pltpu.VMEM((BE, HID), jnp.bfloat16),
            pltpu.VMEM((BE, WD), jnp.float32),
            pltpu.VMEM((BE, WD), jnp.float32),
            pltpu.VMEM((HID,), jnp.float32),
            pltpu.VMEM((ZR, WD), jnp.float32),
            pltpu.VMEM_SHARED((N, WD), jnp.float32),
            pltpu.SemaphoreType.DMA,
            pltpu.SemaphoreType.DMA,
            pltpu.SemaphoreType.DMA,
            pltpu.SemaphoreType.DMA,
        ],
    )
    def k(xl_hbm, xr_hbm, src_hbm, dst_hbm, att_hbm, out_hbm,
          src_v, dst_v, gl0, gd0, gl1, gd1, sb0, sb1, att_v, zb, acc,
          sg0, sg1, ss0, ss1):
        c = lax.axis_index("c")
        s = lax.axis_index("s")
        wid = c * NS + s
        pltpu.sync_copy(src_hbm.at[wid], src_v)
        pltpu.sync_copy(dst_hbm.at[wid], dst_v)

        zeros16 = jnp.zeros((16,), jnp.float32)

        @pl.loop(0, ZR)
        def _(r):
            for v in range(WD // 16):
                zb[r, pl.ds(v * 16, 16)] = zeros16

        iota = lax.iota(jnp.int32, 16)
        mask0 = jnp.where(iota == 0, 1.0, 0.0).astype(jnp.float32)

        def issue_gathers(g, j, gl, gd, sem):
            pltpu.async_copy(xl_hbm.at[g].at[src_v.at[j]], gl, sem)
            pltpu.async_copy(xr_hbm.at[g].at[dst_v.at[j]], gd, sem)

        def wait_gathers(gl, gd, sem):
            pltpu.make_async_copy(xl_hbm.at[0].at[pl.ds(0, BE)], gl,
                                  sem).wait()
            pltpu.make_async_copy(xl_hbm.at[0].at[pl.ds(0, BE)], gd,
                                  sem).wait()

        def wait_scatter(sb, sem):
            pltpu.make_async_copy(out_hbm.at[0].at[0].at[pl.ds(0, BE)], sb,
                                  sem).wait()

        def compute_block(j, gl, gd, sb, atv):
            @plsc.parallel_loop(0, BE, 1, unroll=25)
            def _(e):
                glv = []
                gdv = []
                for half in range(2):
                    x0, x1 = plsc.unpack(
                        gl[e, pl.ds(half * 32, 32)],
                        format=plsc.PackFormat.INTERLEAVED,
                        preferred_element_type=jnp.float32)
                    glv += [x0, x1]
                    y0, y1 = plsc.unpack(
                        gd[e, pl.ds(half * 32, 32)],
                        format=plsc.PackFormat.INTERLEAVED,
                        preferred_element_type=jnp.float32)
                    gdv += [y0, y1]
                th = None
                for v in range(4):
                    m = glv[v] + gdv[v]
                    t = jnp.maximum(m, 0.2 * m) * atv[v]
                    th = t if th is None else th + t
                ee = jnp.exp(jnp.broadcast_to(jnp.sum(th), (16,)))
                for v in range(4):
                    sb[e, pl.ds(v * 16, 16)] = glv[v] * ee
                sb[e, pl.ds(HID, 16)] = ee * mask0

        for g in range(ngroups):
            pltpu.sync_copy(att_hbm.at[g], att_v)
            atv = [att_v[pl.ds(v * 16, 16)] for v in range(4)]

            off = pl.multiple_of(s * ST, 8)
            for z in range(CH // ZR):
                pltpu.sync_copy(zb, acc.at[pl.ds(off + z * ZR, ZR)])
            plsc.subcore_barrier()

            issue_gathers(g, 0, gl0, gd0, sg0)

            @pl.loop(0, NB // 2)
            def _(t):
                j0 = 2 * t
                wait_gathers(gl0, gd0, sg0)
                issue_gathers(g, j0 + 1, gl1, gd1, sg1)

                @pl.when(t > 0)
                def _():
                    wait_scatter(sb0, ss0)

                compute_block(j0, gl0, gd0, sb0, atv)
                pltpu.async_copy(sb0, acc.at[dst_v.at[j0]], ss0,
                                 add=True)

                @pl.when(t < NB // 2 - 1)
                def _():
                    issue_gathers(g, j0 + 2, gl0, gd0, sg0)

                wait_gathers(gl1, gd1, sg1)

                @pl.when(t > 0)
                def _():
                    wait_scatter(sb1, ss1)

                compute_block(j0 + 1, gl1, gd1, sb1, atv)
                pltpu.async_copy(sb1, acc.at[dst_v.at[j0 + 1]], ss1,
                                 add=True)

            wait_scatter(sb0, ss0)
            wait_scatter(sb1, ss1)

            plsc.subcore_barrier()
            off2 = pl.multiple_of(s * ST, 8)
            pltpu.sync_copy(acc.at[pl.ds(off2, CH)],
                            out_hbm.at[g].at[c].at[pl.ds(off2, CH)])

    return k


_edge_pass_l1 = _edge_pass(H1)
_edge_pass_l23 = _edge_pass(1)


def _make_fin_kernel(heads):
    def _fin_kernel(parts_ref, b_ref, o_ref):
        cols = []
        for g in range(heads):
            blk = parts_ref[g, 0] + parts_ref[g, 1]
            den = blk[:, HID:HID + 1] + 1e-16
            cols.append(blk[:, :HID] / den)
        h = cols[0] if heads == 1 else jnp.concatenate(cols, axis=1)
        h = h + b_ref[...]
        o_ref[...] = jnp.where(h > 0, h, jnp.exp(h) - 1.0)
    return _fin_kernel


def _finalize(parts, b, heads):
    bm = 1000
    return pl.pallas_call(
        _make_fin_kernel(heads),
        grid=(N // bm,),
        in_specs=[pl.BlockSpec((heads, NC, bm, WD), lambda i: (0, 0, i, 0)),
                  pl.BlockSpec((1, heads * HID), lambda i: (0, 0))],
        out_specs=pl.BlockSpec((bm, heads * HID), lambda i: (i, 0)),
        out_shape=jax.ShapeDtypeStruct((N, heads * HID), jnp.float32),
    )(parts, b.reshape(1, -1))


def _pool_kernel(h_ref, b_ref, w_ref, bias_ref, o_ref, accp, accc):
    i = pl.program_id(0)

    @pl.when(i == 0)
    def _():
        accp[...] = jnp.zeros_like(accp)
        accc[...] = jnp.zeros_like(accc)

    gids = lax.broadcasted_iota(jnp.int32, (1, G), 1).astype(jnp.float32)
    onehot = (b_ref[...] == gids).astype(jnp.float32)
    dims = (((0,), (0,)), ((), ()))
    accp[...] += lax.dot_general(onehot, h_ref[...], dims,
                                 preferred_element_type=jnp.float32)
    accc[...] += lax.dot_general(onehot, jnp.ones_like(h_ref[...]), dims,
                                 preferred_element_type=jnp.float32)

    @pl.when(i == pl.num_programs(0) - 1)
    def _():
        pooled = accp[...] / jnp.maximum(accc[...], 1.0)
        o_ref[...] = jnp.dot(pooled, w_ref[...],
                             preferred_element_type=jnp.float32) + bias_ref[...]


def _pool_fc(h, batchf, wfc_pad, bfc_pad):
    bm = 1000
    return pl.pallas_call(
        _pool_kernel,
        grid=(N // bm,),
        in_specs=[pl.BlockSpec((bm, HID), lambda i: (i, 0)),
                  pl.BlockSpec((bm, 1), lambda i: (i, 0)),
                  pl.BlockSpec((HID, 128), lambda i: (0, 0)),
                  pl.BlockSpec((1, 128), lambda i: (0, 0))],
        out_specs=pl.BlockSpec((G, 128), lambda i: (0, 0)),
        out_shape=jax.ShapeDtypeStruct((G, 128), jnp.float32),
        scratch_shapes=[pltpu.VMEM((G, HID), jnp.float32),
                        pltpu.VMEM((G, HID), jnp.float32)],
    )(h, batchf, wfc_pad, bfc_pad)


def kernel(x, edge_index, batch, W1l, W1r, a1, b1, W2l, W2r, a2, b2, W3l,
           W3r, a3, b3, Wfc, bfc):
    src2 = edge_index[0].reshape(NW, IR, BE)
    dst2 = edge_index[1].reshape(NW, IR, BE)

    xl1 = _pallas_matmul_grouped(x, _permute_head_cols(W1l, H1), H1)
    xr1 = _pallas_matmul_grouped(x, _permute_head_cols(W1r, H1), H1)
    parts1 = _edge_pass_l1(xl1, xr1, src2, dst2, a1)
    h = _finalize(parts1, b1, H1)

    xl2 = _pallas_matmul_grouped(h, _permute_head_cols(W2l, 1), 1)
    xr2 = _pallas_matmul_grouped(h, _permute_head_cols(W2r, 1), 1)
    parts2 = _edge_pass_l23(xl2, xr2, src2, dst2, a2)
    h = _finalize(parts2, b2, 1)

    xl3 = _pallas_matmul_grouped(h, _permute_head_cols(W3l, 1), 1)
    xr3 = _pallas_matmul_grouped(h, _permute_head_cols(W3r, 1), 1)
    parts3 = _edge_pass_l23(xl3, xr3, src2, dst2, a3)
    h = _finalize(parts3, b3, 1)

    batchf = batch.astype(jnp.float32).reshape(N, 1)
    wfc_pad = jnp.pad(Wfc, ((0, 0), (0, 128 - OUT)))
    bfc_pad = jnp.pad(bfc, (0, 128 - OUT)).reshape(1, 128)
    out = _pool_fc(h, batchf, wfc_pad, bfc_pad)
    return out[:, :OUT]


# speedup vs baseline: 1.0475x; 1.0475x over previous
import dataclasses
import functools

import jax
import jax.numpy as jnp
from jax import lax
from jax.experimental import pallas as pl
from jax.experimental.pallas import tpu as pltpu
from jax.experimental.pallas import tpu_sc as plsc

N = 10000
E = 320000
DIN = 128
HID = 64
H1 = 8
OUT = 10
G = 64

NC = 2
NS = 16
NW = NC * NS
BE = 125
EPW = E // NW
NB = EPW // BE
IR = EPW // BE
WD = 80
CH = 640
ST = 624
ZR = 32


_PERM64 = [c for chunk in (0, 32) for i in range(16)
           for c in (chunk + i, chunk + 16 + i)]


def _permute_head_cols(w, heads):
    k = w.shape[0]
    wr = w.reshape(k, heads, 64)
    return wr[:, :, jnp.array(_PERM64, jnp.int32)].reshape(k, heads * 64)


def _make_matmul_grouped_kernel(ngroups, bn):
    def _kernel(x_ref, w_ref, o_ref):
        res = jnp.dot(x_ref[...], w_ref[...],
                      preferred_element_type=jnp.float32)
        for g in range(ngroups):
            o_ref[g] = res[:, g * bn:(g + 1) * bn].astype(jnp.bfloat16)
    return _kernel


def _pallas_matmul_grouped(x, w, ngroups):
    m, k = x.shape
    _, n = w.shape
    bn = n // ngroups
    bm = 2000
    return pl.pallas_call(
        _make_matmul_grouped_kernel(ngroups, bn),
        grid=(m // bm,),
        in_specs=[pl.BlockSpec((bm, k), lambda i: (i, 0)),
                  pl.BlockSpec((k, n), lambda i: (0, 0))],
        out_specs=pl.BlockSpec((ngroups, bm, bn), lambda i: (0, i, 0)),
        out_shape=jax.ShapeDtypeStruct((ngroups, m, bn), jnp.bfloat16),
    )(x, w)


def _edge_pass(ngroups):
    mesh = plsc.VectorSubcoreMesh(core_axis_name="c", subcore_axis_name="s")
    cp = pltpu.CompilerParams()
    if "needs_layout_passes" in pltpu.CompilerParams.__dataclass_fields__:
        cp = dataclasses.replace(cp, needs_layout_passes=False)
    if "use_tc_tiling_on_sc" in pltpu.CompilerParams.__dataclass_fields__:
        cp = dataclasses.replace(cp, use_tc_tiling_on_sc=False)

    @functools.partial(
        pl.kernel,
        compiler_params=cp,
        out_type=jax.ShapeDtypeStruct((ngroups, NC, N, WD), jnp.float32),
        mesh=mesh,
        scratch_types=[
            pltpu.VMEM((IR, BE), jnp.int32),
            pltpu.VMEM((IR, BE), jnp.int32),
            pltpu.VMEM((BE, HID), jnp.bfloat16),
            pltpu.VMEM((BE, HID), jnp.bfloat16),
            pltpu.VMEM((BE, HID), jnp.bfloat16),
            pltpu.VMEM((BE, HID), jnp.bfloat16),
            pltpu.VMEM((BE, WD), jnp.float32),
            pltpu.VMEM((BE, WD), jnp.float32),
            pltpu.VMEM((HID,), jnp.float32),
            pltpu.VMEM((ZR, WD), jnp.float32),
            pltpu.VMEM_SHARED((N, WD), jnp.float32),
            pltpu.SemaphoreType.DMA,
            pltpu.SemaphoreType.DMA,
            pltpu.SemaphoreType.DMA,
            pltpu.SemaphoreType.DMA,
        ],
    )
    def k(xl_hbm, xr_hbm, src_hbm, dst_hbm, att_hbm, out_hbm,
          src_v, dst_v, gl0, gd0, gl1, gd1, sb0, sb1, att_v, zb, acc,
          sg0, sg1, ss0, ss1):
        c = lax.axis_index("c")
        s = lax.axis_index("s")
        wid = c * NS + s
        pltpu.sync_copy(src_hbm.at[wid], src_v)
        pltpu.sync_copy(dst_hbm.at[wid], dst_v)

        zeros16 = jnp.zeros((16,), jnp.float32)

        @pl.loop(0, ZR)
        def _(r):
            for v in range(WD // 16):
                zb[r, pl.ds(v * 16, 16)] = zeros16

        iota = lax.iota(jnp.int32, 16)
        mask0 = jnp.where(iota == 0, 1.0, 0.0).astype(jnp.float32)

        def issue_gathers(g, j, gl, gd, sem):
            pltpu.async_copy(xl_hbm.at[g].at[src_v.at[j]], gl, sem)
            pltpu.async_copy(xr_hbm.at[g].at[dst_v.at[j]], gd, sem)

        def wait_gathers(gl, gd, sem):
            pltpu.make_async_copy(xl_hbm.at[0].at[pl.ds(0, BE)], gl,
                                  sem).wait()
            pltpu.make_async_copy(xl_hbm.at[0].at[pl.ds(0, BE)], gd,
                                  sem).wait()

        def wait_scatter(sb, sem):
            pltpu.make_async_copy(out_hbm.at[0].at[0].at[pl.ds(0, BE)], sb,
                                  sem).wait()

        def compute_block(j, gl, gd, sb, atv):
            @plsc.parallel_loop(0, BE, 1, unroll=5)
            def _(e):
                glv = []
                gdv = []
                for half in range(2):
                    x0, x1 = plsc.unpack(
                        gl[e, pl.ds(half * 32, 32)],
                        format=plsc.PackFormat.INTERLEAVED,
                        preferred_element_type=jnp.float32)
                    glv += [x0, x1]
                    y0, y1 = plsc.unpack(
                        gd[e, pl.ds(half * 32, 32)],
                        format=plsc.PackFormat.INTERLEAVED,
                        preferred_element_type=jnp.float32)
                    gdv += [y0, y1]
                th = None
                for v in range(4):
                    m = glv[v] + gdv[v]
                    t = jnp.maximum(m, 0.2 * m) * atv[v]
                    th = t if th is None else th + t
                ee = jnp.exp(jnp.broadcast_to(jnp.sum(th), (16,)))
                for v in range(4):
                    sb[e, pl.ds(v * 16, 16)] = glv[v] * ee
                sb[e, pl.ds(HID, 16)] = ee * mask0

        for g in range(ngroups):
            pltpu.sync_copy(att_hbm.at[g], att_v)
            atv = [att_v[pl.ds(v * 16, 16)] for v in range(4)]

            off = pl.multiple_of(s * ST, 8)
            for z in range(CH // ZR):
                pltpu.sync_copy(zb, acc.at[pl.ds(off + z * ZR, ZR)])
            plsc.subcore_barrier()

            issue_gathers(g, 0, gl0, gd0, sg0)

            @pl.loop(0, NB // 2)
            def _(t):
                j0 = 2 * t
                wait_gathers(gl0, gd0, sg0)
                issue_gathers(g, j0 + 1, gl1, gd1, sg1)

                @pl.when(t > 0)
                def _():
                    wait_scatter(sb0, ss0)

                compute_block(j0, gl0, gd0, sb0, atv)
                pltpu.async_copy(sb0, acc.at[dst_v.at[j0]], ss0,
                                 add=True)

                @pl.when(t < NB // 2 - 1)
                def _():
                    issue_gathers(g, j0 + 2, gl0, gd0, sg0)

                wait_gathers(gl1, gd1, sg1)

                @pl.when(t > 0)
                def _():
                    wait_scatter(sb1, ss1)

                compute_block(j0 + 1, gl1, gd1, sb1, atv)
                pltpu.async_copy(sb1, acc.at[dst_v.at[j0 + 1]], ss1,
                                 add=True)

            wait_scatter(sb0, ss0)
            wait_scatter(sb1, ss1)

            plsc.subcore_barrier()
            off2 = pl.multiple_of(s * ST, 8)
            pltpu.sync_copy(acc.at[pl.ds(off2, CH)],
                            out_hbm.at[g].at[c].at[pl.ds(off2, CH)])

    return k


_edge_pass_l1 = _edge_pass(H1)
_edge_pass_l23 = _edge_pass(1)


def _make_fin_kernel(heads):
    def _fin_kernel(parts_ref, b_ref, o_ref):
        cols = []
        for g in range(heads):
            blk = parts_ref[g, 0] + parts_ref[g, 1]
            den = blk[:, HID:HID + 1] + 1e-16
            cols.append(blk[:, :HID] / den)
        h = cols[0] if heads == 1 else jnp.concatenate(cols, axis=1)
        h = h + b_ref[...]
        o_ref[...] = jnp.where(h > 0, h, jnp.exp(h) - 1.0)
    return _fin_kernel


def _finalize(parts, b, heads):
    bm = 1000
    return pl.pallas_call(
        _make_fin_kernel(heads),
        grid=(N // bm,),
        in_specs=[pl.BlockSpec((heads, NC, bm, WD), lambda i: (0, 0, i, 0)),
                  pl.BlockSpec((1, heads * HID), lambda i: (0, 0))],
        out_specs=pl.BlockSpec((bm, heads * HID), lambda i: (i, 0)),
        out_shape=jax.ShapeDtypeStruct((N, heads * HID), jnp.float32),
    )(parts, b.reshape(1, -1))


def _pool_kernel(h_ref, b_ref, w_ref, bias_ref, o_ref, accp, accc):
    i = pl.program_id(0)

    @pl.when(i == 0)
    def _():
        accp[...] = jnp.zeros_like(accp)
        accc[...] = jnp.zeros_like(accc)

    gids = lax.broadcasted_iota(jnp.int32, (1, G), 1).astype(jnp.float32)
    onehot = (b_ref[...] == gids).astype(jnp.float32)
    dims = (((0,), (0,)), ((), ()))
    accp[...] += lax.dot_general(onehot, h_ref[...], dims,
                                 preferred_element_type=jnp.float32)
    accc[...] += lax.dot_general(onehot, jnp.ones_like(h_ref[...]), dims,
                                 preferred_element_type=jnp.float32)

    @pl.when(i == pl.num_programs(0) - 1)
    def _():
        pooled = accp[...] / jnp.maximum(accc[...], 1.0)
        o_ref[...] = jnp.dot(pooled, w_ref[...],
                             preferred_element_type=jnp.float32) + bias_ref[...]


def _pool_fc(h, batchf, wfc_pad, bfc_pad):
    bm = 1000
    return pl.pallas_call(
        _pool_kernel,
        grid=(N // bm,),
        in_specs=[pl.BlockSpec((bm, HID), lambda i: (i, 0)),
                  pl.BlockSpec((bm, 1), lambda i: (i, 0)),
                  pl.BlockSpec((HID, 128), lambda i: (0, 0)),
                  pl.BlockSpec((1, 128), lambda i: (0, 0))],
        out_specs=pl.BlockSpec((G, 128), lambda i: (0, 0)),
        out_shape=jax.ShapeDtypeStruct((G, 128), jnp.float32),
        scratch_shapes=[pltpu.VMEM((G, HID), jnp.float32),
                        pltpu.VMEM((G, HID), jnp.float32)],
    )(h, batchf, wfc_pad, bfc_pad)


def kernel(x, edge_index, batch, W1l, W1r, a1, b1, W2l, W2r, a2, b2, W3l,
           W3r, a3, b3, Wfc, bfc):
    src2 = edge_index[0].reshape(NW, IR, BE)
    dst2 = edge_index[1].reshape(NW, IR, BE)

    xl1 = _pallas_matmul_grouped(x, _permute_head_cols(W1l, H1), H1)
    xr1 = _pallas_matmul_grouped(x, _permute_head_cols(W1r, H1), H1)
    parts1 = _edge_pass_l1(xl1, xr1, src2, dst2, a1)
    h = _finalize(parts1, b1, H1)

    xl2 = _pallas_matmul_grouped(h, _permute_head_cols(W2l, 1), 1)
    xr2 = _pallas_matmul_grouped(h, _permute_head_cols(W2r, 1), 1)
    parts2 = _edge_pass_l23(xl2, xr2, src2, dst2, a2)
    h = _finalize(parts2, b2, 1)

    xl3 = _pallas_matmul_grouped(h, _permute_head_cols(W3l, 1), 1)
    xr3 = _pallas_matmul_grouped(h, _permute_head_cols(W3r, 1), 1)
    parts3 = _edge_pass_l23(xl3, xr3, src2, dst2, a3)
    h = _finalize(parts3, b3, 1)

    batchf = batch.astype(jnp.float32).reshape(N, 1)
    wfc_pad = jnp.pad(Wfc, ((0, 0), (0, 128 - OUT)))
    bfc_pad = jnp.pad(bfc, (0, 128 - OUT)).reshape(1, 128)
    out = _pool_fc(h, batchf, wfc_pad, bfc_pad)
    return out[:, :OUT]


# speedup vs baseline: 1.0935x; 1.0439x over previous
import dataclasses
import functools

import jax
import jax.numpy as jnp
from jax import lax
from jax.experimental import pallas as pl
from jax.experimental.pallas import tpu as pltpu
from jax.experimental.pallas import tpu_sc as plsc

N = 10000
E = 320000
DIN = 128
HID = 64
H1 = 8
OUT = 10
G = 64

NC = 2
NS = 16
NW = NC * NS
BE = 125
EPW = E // NW
NB = EPW // BE
IR = EPW // BE
WD = 80
CH = 640
ST = 624
ZR = 32


_PERM64 = [c for chunk in (0, 32) for i in range(16)
           for c in (chunk + i, chunk + 16 + i)]


def _permute_head_cols(w, heads):
    k = w.shape[0]
    wr = w.reshape(k, heads, 64)
    return wr[:, :, jnp.array(_PERM64, jnp.int32)].reshape(k, heads * 64)


def _make_dual_matmul_kernel(ngroups):
    def _kernel(x_ref, wl_ref, wr_ref, ol_ref, or_ref):
        xb = x_ref[...]
        resl = jnp.dot(xb, wl_ref[...], preferred_element_type=jnp.float32)
        resr = jnp.dot(xb, wr_ref[...], preferred_element_type=jnp.float32)
        for g in range(ngroups):
            ol_ref[g] = resl[:, g * HID:(g + 1) * HID].astype(jnp.bfloat16)
            or_ref[g] = resr[:, g * HID:(g + 1) * HID].astype(jnp.bfloat16)
    return _kernel


def _dual_matmul(x, wl, wr, ngroups):
    m, k = x.shape
    n = wl.shape[1]
    bm = 2000
    tab = jax.ShapeDtypeStruct((ngroups, m, HID), jnp.bfloat16)
    return pl.pallas_call(
        _make_dual_matmul_kernel(ngroups),
        grid=(m // bm,),
        in_specs=[pl.BlockSpec((bm, k), lambda i: (i, 0)),
                  pl.BlockSpec((k, n), lambda i: (0, 0)),
                  pl.BlockSpec((k, n), lambda i: (0, 0))],
        out_specs=[pl.BlockSpec((ngroups, bm, HID), lambda i: (0, i, 0)),
                   pl.BlockSpec((ngroups, bm, HID), lambda i: (0, i, 0))],
        out_shape=[tab, tab],
    )(x, wl, wr)


def _edge_pass(ngroups):
    mesh = plsc.VectorSubcoreMesh(core_axis_name="c", subcore_axis_name="s")
    cp = pltpu.CompilerParams()
    if "needs_layout_passes" in pltpu.CompilerParams.__dataclass_fields__:
        cp = dataclasses.replace(cp, needs_layout_passes=False)
    if "use_tc_tiling_on_sc" in pltpu.CompilerParams.__dataclass_fields__:
        cp = dataclasses.replace(cp, use_tc_tiling_on_sc=False)

    @functools.partial(
        pl.kernel,
        compiler_params=cp,
        out_type=jax.ShapeDtypeStruct((ngroups, NC, N, WD), jnp.float32),
        mesh=mesh,
        scratch_types=[
            pltpu.VMEM((IR, BE), jnp.int32),
            pltpu.VMEM((IR, BE), jnp.int32),
            pltpu.VMEM((BE, HID), jnp.bfloat16),
            pltpu.VMEM((BE, HID), jnp.bfloat16),
            pltpu.VMEM((BE, HID), jnp.bfloat16),
            pltpu.VMEM((BE, HID), jnp.bfloat16),
            pltpu.VMEM((BE, WD), jnp.float32),
            pltpu.VMEM((BE, WD), jnp.float32),
            pltpu.VMEM((HID,), jnp.float32),
            pltpu.VMEM((ZR, WD), jnp.float32),
            pltpu.VMEM_SHARED((N, WD), jnp.float32),
            pltpu.SemaphoreType.DMA,
            pltpu.SemaphoreType.DMA,
            pltpu.SemaphoreType.DMA,
            pltpu.SemaphoreType.DMA,
        ],
    )
    def k(xl_hbm, xr_hbm, src_hbm, dst_hbm, att_hbm, out_hbm,
          src_v, dst_v, gl0, gd0, gl1, gd1, sb0, sb1, att_v, zb, acc,
          sg0, sg1, ss0, ss1):
        c = lax.axis_index("c")
        s = lax.axis_index("s")
        wid = c * NS + s
        pltpu.sync_copy(src_hbm.at[wid], src_v)
        pltpu.sync_copy(dst_hbm.at[wid], dst_v)

        zeros16 = jnp.zeros((16,), jnp.float32)

        @pl.loop(0, ZR)
        def _(r):
            for v in range(WD // 16):
                zb[r, pl.ds(v * 16, 16)] = zeros16

        iota = lax.iota(jnp.int32, 16)
        mask0 = jnp.where(iota == 0, 1.0, 0.0).astype(jnp.float32)

        def issue_gathers(g, j, gl, gd, sem):
            pltpu.async_copy(xl_hbm.at[g].at[src_v.at[j]], gl, sem)
            pltpu.async_copy(xr_hbm.at[g].at[dst_v.at[j]], gd, sem)

        def wait_gathers(gl, gd, sem):
            pltpu.make_async_copy(xl_hbm.at[0].at[pl.ds(0, BE)], gl,
                                  sem).wait()
            pltpu.make_async_copy(xl_hbm.at[0].at[pl.ds(0, BE)], gd,
                                  sem).wait()

        def wait_scatter(sb, sem):
            pltpu.make_async_copy(out_hbm.at[0].at[0].at[pl.ds(0, BE)], sb,
                                  sem).wait()

        def compute_block(j, gl, gd, sb, atv):
            @plsc.parallel_loop(0, BE, 1, unroll=5)
            def _(e):
                glv = []
                gdv = []
                for half in range(2):
                    x0, x1 = plsc.unpack(
                        gl[e, pl.ds(half * 32, 32)],
                        format=plsc.PackFormat.INTERLEAVED,
                        preferred_element_type=jnp.float32)
                    glv += [x0, x1]
                    y0, y1 = plsc.unpack(
                        gd[e, pl.ds(half * 32, 32)],
                        format=plsc.PackFormat.INTERLEAVED,
                        preferred_element_type=jnp.float32)
                    gdv += [y0, y1]
                th = None
                for v in range(4):
                    m = glv[v] + gdv[v]
                    t = jnp.maximum(m, 0.2 * m) * atv[v]
                    th = t if th is None else th + t
                ee = jnp.exp(jnp.broadcast_to(jnp.sum(th), (16,)))
                for v in range(4):
                    sb[e, pl.ds(v * 16, 16)] = glv[v] * ee
                sb[e, pl.ds(HID, 16)] = ee * mask0

        for g in range(ngroups):
            pltpu.sync_copy(att_hbm.at[g], att_v)
            atv = [att_v[pl.ds(v * 16, 16)] for v in range(4)]

            off = pl.multiple_of(s * ST, 8)
            for z in range(CH // ZR):
                pltpu.sync_copy(zb, acc.at[pl.ds(off + z * ZR, ZR)])
            plsc.subcore_barrier()

            issue_gathers(g, 0, gl0, gd0, sg0)

            @pl.loop(0, NB // 2)
            def _(t):
                j0 = 2 * t
                wait_gathers(gl0, gd0, sg0)
                issue_gathers(g, j0 + 1, gl1, gd1, sg1)

                @pl.when(t > 0)
                def _():
                    wait_scatter(sb0, ss0)

                compute_block(j0, gl0, gd0, sb0, atv)
                pltpu.async_copy(sb0, acc.at[dst_v.at[j0]], ss0,
                                 add=True)

                @pl.when(t < NB // 2 - 1)
                def _():
                    issue_gathers(g, j0 + 2, gl0, gd0, sg0)

                wait_gathers(gl1, gd1, sg1)

                @pl.when(t > 0)
                def _():
                    wait_scatter(sb1, ss1)

                compute_block(j0 + 1, gl1, gd1, sb1, atv)
                pltpu.async_copy(sb1, acc.at[dst_v.at[j0 + 1]], ss1,
                                 add=True)

            wait_scatter(sb0, ss0)
            wait_scatter(sb1, ss1)

            plsc.subcore_barrier()
            off2 = pl.multiple_of(s * ST, 8)
            pltpu.sync_copy(acc.at[pl.ds(off2, CH)],
                            out_hbm.at[g].at[c].at[pl.ds(off2, CH)])

    return k


_edge_pass_l1 = _edge_pass(H1)
_edge_pass_l23 = _edge_pass(1)


def _fin_h(parts_ref, b_ref, heads):
    cols = []
    for g in range(heads):
        blk = parts_ref[g, 0] + parts_ref[g, 1]
        den = blk[:, HID:HID + 1] + 1e-16
        cols.append(blk[:, :HID] / den)
    h = cols[0] if heads == 1 else jnp.concatenate(cols, axis=1)
    h = h + b_ref[...]
    return jnp.where(h > 0, h, jnp.exp(h) - 1.0)


def _make_fin_mm_kernel(heads, ngroups_out):
    def _kernel(parts_ref, b_ref, wl_ref, wr_ref, ol_ref, or_ref):
        h = _fin_h(parts_ref, b_ref, heads)
        resl = jnp.dot(h, wl_ref[...], preferred_element_type=jnp.float32)
        resr = jnp.dot(h, wr_ref[...], preferred_element_type=jnp.float32)
        for g in range(ngroups_out):
            ol_ref[g] = resl[:, g * HID:(g + 1) * HID].astype(jnp.bfloat16)
            or_ref[g] = resr[:, g * HID:(g + 1) * HID].astype(jnp.bfloat16)
    return _kernel


def _finalize_mm(parts, b, heads, wl, wr):
    bm = 2000
    n_out = wl.shape[1]
    ngroups_out = n_out // HID
    tab = jax.ShapeDtypeStruct((ngroups_out, N, HID), jnp.bfloat16)
    return pl.pallas_call(
        _make_fin_mm_kernel(heads, ngroups_out),
        grid=(N // bm,),
        in_specs=[pl.BlockSpec((heads, NC, bm, WD), lambda i: (0, 0, i, 0)),
                  pl.BlockSpec((1, heads * HID), lambda i: (0, 0)),
                  pl.BlockSpec((heads * HID, n_out), lambda i: (0, 0)),
                  pl.BlockSpec((heads * HID, n_out), lambda i: (0, 0))],
        out_specs=[pl.BlockSpec((ngroups_out, bm, HID), lambda i: (0, i, 0)),
                   pl.BlockSpec((ngroups_out, bm, HID), lambda i: (0, i, 0))],
        out_shape=[tab, tab],
    )(parts, b.reshape(1, -1), wl, wr)


def _fin_pool_kernel(parts_ref, b_ref, batch_ref, w_ref, bias_ref, o_ref,
                     accp, accc):
    i = pl.program_id(0)

    @pl.when(i == 0)
    def _():
        accp[...] = jnp.zeros_like(accp)
        accc[...] = jnp.zeros_like(accc)

    h = _fin_h(parts_ref, b_ref, 1)
    gids = lax.broadcasted_iota(jnp.int32, (1, G), 1).astype(jnp.float32)
    onehot = (batch_ref[...] == gids).astype(jnp.float32)
    dims = (((0,), (0,)), ((), ()))
    accp[...] += lax.dot_general(onehot, h, dims,
                                 preferred_element_type=jnp.float32)
    accc[...] += lax.dot_general(onehot, jnp.ones_like(h), dims,
                                 preferred_element_type=jnp.float32)

    @pl.when(i == pl.num_programs(0) - 1)
    def _():
        pooled = accp[...] / jnp.maximum(accc[...], 1.0)
        o_ref[...] = jnp.dot(pooled, w_ref[...],
                             preferred_element_type=jnp.float32) + bias_ref[...]


def _finalize_pool_fc(parts, b, batchf, wfc_pad, bfc_pad):
    bm = 2000
    return pl.pallas_call(
        _fin_pool_kernel,
        grid=(N // bm,),
        in_specs=[pl.BlockSpec((1, NC, bm, WD), lambda i: (0, 0, i, 0)),
                  pl.BlockSpec((1, HID), lambda i: (0, 0)),
                  pl.BlockSpec((bm, 1), lambda i: (i, 0)),
                  pl.BlockSpec((HID, 128), lambda i: (0, 0)),
                  pl.BlockSpec((1, 128), lambda i: (0, 0))],
        out_specs=pl.BlockSpec((G, 128), lambda i: (0, 0)),
        out_shape=jax.ShapeDtypeStruct((G, 128), jnp.float32),
        scratch_shapes=[pltpu.VMEM((G, HID), jnp.float32),
                        pltpu.VMEM((G, HID), jnp.float32)],
    )(parts, b.reshape(1, -1), batchf, wfc_pad, bfc_pad)


def kernel(x, edge_index, batch, W1l, W1r, a1, b1, W2l, W2r, a2, b2, W3l,
           W3r, a3, b3, Wfc, bfc):
    src2 = edge_index[0].reshape(NW, IR, BE)
    dst2 = edge_index[1].reshape(NW, IR, BE)

    xl1, xr1 = _dual_matmul(x, _permute_head_cols(W1l, H1),
                            _permute_head_cols(W1r, H1), H1)
    parts1 = _edge_pass_l1(xl1, xr1, src2, dst2, a1)

    xl2, xr2 = _finalize_mm(parts1, b1, H1, _permute_head_cols(W2l, 1),
                            _permute_head_cols(W2r, 1))
    parts2 = _edge_pass_l23(xl2, xr2, src2, dst2, a2)

    xl3, xr3 = _finalize_mm(parts2, b2, 1, _permute_head_cols(W3l, 1),
                            _permute_head_cols(W3r, 1))
    parts3 = _edge_pass_l23(xl3, xr3, src2, dst2, a3)

    batchf = batch.astype(jnp.float32).reshape(N, 1)
    wfc_pad = jnp.pad(Wfc, ((0, 0), (0, 128 - OUT)))
    bfc_pad = jnp.pad(bfc, (0, 128 - OUT)).reshape(1, 128)
    out = _finalize_pool_fc(parts3, b3, batchf, wfc_pad, bfc_pad)
    return out[:, :OUT]


# speedup vs baseline: 1.1020x; 1.0077x over previous
import dataclasses
import functools

import jax
import jax.numpy as jnp
from jax import lax
from jax.experimental import pallas as pl
from jax.experimental.pallas import tpu as pltpu
from jax.experimental.pallas import tpu_sc as plsc

N = 10000
E = 320000
DIN = 128
HID = 64
H1 = 8
OUT = 10
G = 64

NC = 2
NS = 16
NW = NC * NS
BE = 125
EPW = E // NW
NB = EPW // BE
IR = EPW // BE
WD = 80
CH = 640
ST = 624
ZR = 128


_PERM64 = [c for chunk in (0, 32) for i in range(16)
           for c in (chunk + i, chunk + 16 + i)]


def _permute_head_cols(w, heads):
    k = w.shape[0]
    wr = w.reshape(k, heads, 64)
    return wr[:, :, jnp.array(_PERM64, jnp.int32)].reshape(k, heads * 64)


def _make_dual_matmul_kernel(ngroups):
    def _kernel(x_ref, wl_ref, wr_ref, ol_ref, or_ref):
        xb = x_ref[...]
        resl = jnp.dot(xb, wl_ref[...], preferred_element_type=jnp.float32)
        resr = jnp.dot(xb, wr_ref[...], preferred_element_type=jnp.float32)
        for g in range(ngroups):
            ol_ref[g] = resl[:, g * HID:(g + 1) * HID].astype(jnp.bfloat16)
            or_ref[g] = resr[:, g * HID:(g + 1) * HID].astype(jnp.bfloat16)
    return _kernel


def _dual_matmul(x, wl, wr, ngroups):
    m, k = x.shape
    n = wl.shape[1]
    bm = 2000
    tab = jax.ShapeDtypeStruct((ngroups, m, HID), jnp.bfloat16)
    return pl.pallas_call(
        _make_dual_matmul_kernel(ngroups),
        grid=(m // bm,),
        in_specs=[pl.BlockSpec((bm, k), lambda i: (i, 0)),
                  pl.BlockSpec((k, n), lambda i: (0, 0)),
                  pl.BlockSpec((k, n), lambda i: (0, 0))],
        out_specs=[pl.BlockSpec((ngroups, bm, HID), lambda i: (0, i, 0)),
                   pl.BlockSpec((ngroups, bm, HID), lambda i: (0, i, 0))],
        out_shape=[tab, tab],
    )(x, wl, wr)


def _edge_pass(ngroups):
    mesh = plsc.VectorSubcoreMesh(core_axis_name="c", subcore_axis_name="s")
    cp = pltpu.CompilerParams()
    if "needs_layout_passes" in pltpu.CompilerParams.__dataclass_fields__:
        cp = dataclasses.replace(cp, needs_layout_passes=False)
    if "use_tc_tiling_on_sc" in pltpu.CompilerParams.__dataclass_fields__:
        cp = dataclasses.replace(cp, use_tc_tiling_on_sc=False)

    @functools.partial(
        pl.kernel,
        compiler_params=cp,
        out_type=jax.ShapeDtypeStruct((ngroups, NC, N, WD), jnp.float32),
        mesh=mesh,
        scratch_types=[
            pltpu.VMEM((IR, BE), jnp.int32),
            pltpu.VMEM((IR, BE), jnp.int32),
            pltpu.VMEM((BE, HID), jnp.bfloat16),
            pltpu.VMEM((BE, HID), jnp.bfloat16),
            pltpu.VMEM((BE, HID), jnp.bfloat16),
            pltpu.VMEM((BE, HID), jnp.bfloat16),
            pltpu.VMEM((BE, WD), jnp.float32),
            pltpu.VMEM((BE, WD), jnp.float32),
            pltpu.VMEM((HID,), jnp.float32),
            pltpu.VMEM((ZR, WD), jnp.float32),
            pltpu.VMEM_SHARED((N, WD), jnp.float32),
            pltpu.SemaphoreType.DMA,
            pltpu.SemaphoreType.DMA,
            pltpu.SemaphoreType.DMA,
            pltpu.SemaphoreType.DMA,
            pltpu.SemaphoreType.DMA,
        ],
    )
    def k(xl_hbm, xr_hbm, src_hbm, dst_hbm, att_hbm, out_hbm,
          src_v, dst_v, gl0, gd0, gl1, gd1, sb0, sb1, att_v, zb, acc,
          sg0, sg1, ss0, ss1, sz):
        c = lax.axis_index("c")
        s = lax.axis_index("s")
        wid = c * NS + s
        pltpu.sync_copy(src_hbm.at[wid], src_v)
        pltpu.sync_copy(dst_hbm.at[wid], dst_v)

        zeros16 = jnp.zeros((16,), jnp.float32)

        @pl.loop(0, ZR)
        def _(r):
            for v in range(WD // 16):
                zb[r, pl.ds(v * 16, 16)] = zeros16

        iota = lax.iota(jnp.int32, 16)
        mask0 = jnp.where(iota == 0, 1.0, 0.0).astype(jnp.float32)

        def issue_gathers(g, j, gl, gd, sem):
            pltpu.async_copy(xl_hbm.at[g].at[src_v.at[j]], gl, sem)
            pltpu.async_copy(xr_hbm.at[g].at[dst_v.at[j]], gd, sem)

        def wait_gathers(gl, gd, sem):
            pltpu.make_async_copy(xl_hbm.at[0].at[pl.ds(0, BE)], gl,
                                  sem).wait()
            pltpu.make_async_copy(xl_hbm.at[0].at[pl.ds(0, BE)], gd,
                                  sem).wait()

        def wait_scatter(sb, sem):
            pltpu.make_async_copy(out_hbm.at[0].at[0].at[pl.ds(0, BE)], sb,
                                  sem).wait()

        def compute_block(j, gl, gd, sb, atv):
            @plsc.parallel_loop(0, BE, 1, unroll=5)
            def _(e):
                glv = []
                gdv = []
                for half in range(2):
                    x0, x1 = plsc.unpack(
                        gl[e, pl.ds(half * 32, 32)],
                        format=plsc.PackFormat.INTERLEAVED,
                        preferred_element_type=jnp.float32)
                    glv += [x0, x1]
                    y0, y1 = plsc.unpack(
                        gd[e, pl.ds(half * 32, 32)],
                        format=plsc.PackFormat.INTERLEAVED,
                        preferred_element_type=jnp.float32)
                    gdv += [y0, y1]
                th = None
                for v in range(4):
                    m = glv[v] + gdv[v]
                    t = jnp.maximum(m, 0.2 * m) * atv[v]
                    th = t if th is None else th + t
                ee = jnp.exp(jnp.broadcast_to(jnp.sum(th), (16,)))
                for v in range(4):
                    sb[e, pl.ds(v * 16, 16)] = glv[v] * ee
                sb[e, pl.ds(HID, 16)] = ee * mask0

        for g in range(ngroups):
            pltpu.sync_copy(att_hbm.at[g], att_v)
            atv = [att_v[pl.ds(v * 16, 16)] for v in range(4)]

            off = pl.multiple_of(s * ST, 8)
            for z in range(CH // ZR):
                pltpu.async_copy(zb, acc.at[pl.ds(off + z * ZR, ZR)], sz)
            for z in range(CH // ZR):
                pltpu.make_async_copy(out_hbm.at[0].at[0].at[pl.ds(0, ZR)],
                                      zb, sz).wait()
            plsc.subcore_barrier()

            issue_gathers(g, 0, gl0, gd0, sg0)

            @pl.loop(0, NB // 2)
            def _(t):
                j0 = 2 * t
                wait_gathers(gl0, gd0, sg0)
                issue_gathers(g, j0 + 1, gl1, gd1, sg1)

                @pl.when(t > 0)
                def _():
                    wait_scatter(sb0, ss0)

                compute_block(j0, gl0, gd0, sb0, atv)
                pltpu.async_copy(sb0, acc.at[dst_v.at[j0]], ss0,
                                 add=True)

                @pl.when(t < NB // 2 - 1)
                def _():
                    issue_gathers(g, j0 + 2, gl0, gd0, sg0)

                wait_gathers(gl1, gd1, sg1)

                @pl.when(t > 0)
                def _():
                    wait_scatter(sb1, ss1)

                compute_block(j0 + 1, gl1, gd1, sb1, atv)
                pltpu.async_copy(sb1, acc.at[dst_v.at[j0 + 1]], ss1,
                                 add=True)

            wait_scatter(sb0, ss0)
            wait_scatter(sb1, ss1)

            plsc.subcore_barrier()
            off2 = pl.multiple_of(s * ST, 8)
            pltpu.sync_copy(acc.at[pl.ds(off2, CH)],
                            out_hbm.at[g].at[c].at[pl.ds(off2, CH)])

    return k


_edge_pass_l1 = _edge_pass(H1)
_edge_pass_l23 = _edge_pass(1)


def _fin_h(parts_ref, b_ref, heads):
    cols = []
    for g in range(heads):
        blk = parts_ref[g, 0] + parts_ref[g, 1]
        den = blk[:, HID:HID + 1] + 1e-16
        cols.append(blk[:, :HID] / den)
    h = cols[0] if heads == 1 else jnp.concatenate(cols, axis=1)
    h = h + b_ref[...]
    return jnp.where(h > 0, h, jnp.exp(h) - 1.0)


def _make_fin_mm_kernel(heads, ngroups_out):
    def _kernel(parts_ref, b_ref, wl_ref, wr_ref, ol_ref, or_ref):
        h = _fin_h(parts_ref, b_ref, heads)
        resl = jnp.dot(h, wl_ref[...], preferred_element_type=jnp.float32)
        resr = jnp.dot(h, wr_ref[...], preferred_element_type=jnp.float32)
        for g in range(ngroups_out):
            ol_ref[g] = resl[:, g * HID:(g + 1) * HID].astype(jnp.bfloat16)
            or_ref[g] = resr[:, g * HID:(g + 1) * HID].astype(jnp.bfloat16)
    return _kernel


def _finalize_mm(parts, b, heads, wl, wr):
    bm = 2000
    n_out = wl.shape[1]
    ngroups_out = n_out // HID
    tab = jax.ShapeDtypeStruct((ngroups_out, N, HID), jnp.bfloat16)
    return pl.pallas_call(
        _make_fin_mm_kernel(heads, ngroups_out),
        grid=(N // bm,),
        in_specs=[pl.BlockSpec((heads, NC, bm, WD), lambda i: (0, 0, i, 0)),
                  pl.BlockSpec((1, heads * HID), lambda i: (0, 0)),
                  pl.BlockSpec((heads * HID, n_out), lambda i: (0, 0)),
                  pl.BlockSpec((heads * HID, n_out), lambda i: (0, 0))],
        out_specs=[pl.BlockSpec((ngroups_out, bm, HID), lambda i: (0, i, 0)),
                   pl.BlockSpec((ngroups_out, bm, HID), lambda i: (0, i, 0))],
        out_shape=[tab, tab],
    )(parts, b.reshape(1, -1), wl, wr)


def _fin_pool_kernel(parts_ref, b_ref, batch_ref, w_ref, bias_ref, o_ref,
                     accp, accc):
    i = pl.program_id(0)

    @pl.when(i == 0)
    def _():
        accp[...] = jnp.zeros_like(accp)
        accc[...] = jnp.zeros_like(accc)

    h = _fin_h(parts_ref, b_ref, 1)
    gids = lax.broadcasted_iota(jnp.int32, (1, G), 1).astype(jnp.float32)
    onehot = (batch_ref[...] == gids).astype(jnp.float32)
    dims = (((0,), (0,)), ((), ()))
    accp[...] += lax.dot_general(onehot, h, dims,
                                 preferred_element_type=jnp.float32)
    accc[...] += lax.dot_general(onehot, jnp.ones_like(h), dims,
                                 preferred_element_type=jnp.float32)

    @pl.when(i == pl.num_programs(0) - 1)
    def _():
        pooled = accp[...] / jnp.maximum(accc[...], 1.0)
        o_ref[...] = jnp.dot(pooled, w_ref[...],
                             preferred_element_type=jnp.float32) + bias_ref[...]


def _finalize_pool_fc(parts, b, batchf, wfc_pad, bfc_pad):
    bm = 2000
    return pl.pallas_call(
        _fin_pool_kernel,
        grid=(N // bm,),
        in_specs=[pl.BlockSpec((1, NC, bm, WD), lambda i: (0, 0, i, 0)),
                  pl.BlockSpec((1, HID), lambda i: (0, 0)),
                  pl.BlockSpec((bm, 1), lambda i: (i, 0)),
                  pl.BlockSpec((HID, 128), lambda i: (0, 0)),
                  pl.BlockSpec((1, 128), lambda i: (0, 0))],
        out_specs=pl.BlockSpec((G, 128), lambda i: (0, 0)),
        out_shape=jax.ShapeDtypeStruct((G, 128), jnp.float32),
        scratch_shapes=[pltpu.VMEM((G, HID), jnp.float32),
                        pltpu.VMEM((G, HID), jnp.float32)],
    )(parts, b.reshape(1, -1), batchf, wfc_pad, bfc_pad)


def kernel(x, edge_index, batch, W1l, W1r, a1, b1, W2l, W2r, a2, b2, W3l,
           W3r, a3, b3, Wfc, bfc):
    src2 = edge_index[0].reshape(NW, IR, BE)
    dst2 = edge_index[1].reshape(NW, IR, BE)

    xl1, xr1 = _dual_matmul(x, _permute_head_cols(W1l, H1),
                            _permute_head_cols(W1r, H1), H1)
    parts1 = _edge_pass_l1(xl1, xr1, src2, dst2, a1)

    xl2, xr2 = _finalize_mm(parts1, b1, H1, _permute_head_cols(W2l, 1),
                            _permute_head_cols(W2r, 1))
    parts2 = _edge_pass_l23(xl2, xr2, src2, dst2, a2)

    xl3, xr3 = _finalize_mm(parts2, b2, 1, _permute_head_cols(W3l, 1),
                            _permute_head_cols(W3r, 1))
    parts3 = _edge_pass_l23(xl3, xr3, src2, dst2, a3)

    batchf = batch.astype(jnp.float32).reshape(N, 1)
    wfc_pad = jnp.pad(Wfc, ((0, 0), (0, 128 - OUT)))
    bfc_pad = jnp.pad(bfc, (0, 128 - OUT)).reshape(1, 128)
    out = _finalize_pool_fc(parts3, b3, batchf, wfc_pad, bfc_pad)
    return out[:, :OUT]
